# Initial kernel scaffold; baseline (speedup 1.0000x reference)
import jax
import jax.numpy as jnp
from jax.experimental import pallas as pl


def kernel(x, edge_index, W_q, W_k, W_v, W_o, b_o, W_g1, b_g1, W_g2, b_g2):
    raise NotImplementedError("write your pallas kernel here")


# speedup vs baseline: 13.0166x; 13.0166x over previous
import functools

import jax
import jax.numpy as jnp
from jax import lax
from jax.experimental import pallas as pl
from jax.experimental.pallas import tpu as pltpu
from jax.experimental.pallas import tpu_sc as plsc

N_NODES = 10000
N_EDGES = 320000
HIDDEN = 128
HEADS = 4
HEAD_DIM = 32
INV_SQRT_HD = 1.0 / (HEAD_DIM ** 0.5)

NC = 2
NS = 16
NW = NC * NS
EPW = N_EDGES // NW
CHUNK = 80
NCHUNK = EPW // CHUNK
GRP = CHUNK // 16
ROWBLK = 1000


def _prep_body(x_ref, wq, wk, wv, wg1, bg1, wg2, bg2, qg_ref, k_ref, v_ref):
    xb = x_ref[...]
    q = jnp.dot(xb, wq[...], preferred_element_type=jnp.float32)
    k_ref[...] = jnp.dot(xb, wk[...], preferred_element_type=jnp.float32)
    v_ref[...] = jnp.dot(xb, wv[...], preferred_element_type=jnp.float32)
    h1 = jnp.maximum(jnp.dot(xb, wg1[...], preferred_element_type=jnp.float32)
                     + bg1[...], 0.0)
    s = jnp.sum(h1 * wg2[...], axis=1, keepdims=True) + bg2[...]
    g = jax.nn.sigmoid(s)
    qg_ref[...] = q * (g * INV_SQRT_HD)


def _prep(x, wqT, wkT, wvT, wg1T, bg1, wg2, bg2):
    nblk = N_NODES // ROWBLK
    full = lambda i: (0, 0)
    blk = lambda i: (i, 0)
    return pl.pallas_call(
        _prep_body,
        grid=(nblk,),
        in_specs=[
            pl.BlockSpec((ROWBLK, HIDDEN), blk),
            pl.BlockSpec((HIDDEN, HIDDEN), full),
            pl.BlockSpec((HIDDEN, HIDDEN), full),
            pl.BlockSpec((HIDDEN, HIDDEN), full),
            pl.BlockSpec((HIDDEN, HIDDEN // 4), full),
            pl.BlockSpec((1, HIDDEN // 4), full),
            pl.BlockSpec((1, HIDDEN // 4), full),
            pl.BlockSpec((1, 1), full),
        ],
        out_specs=[
            pl.BlockSpec((ROWBLK, HIDDEN), blk),
            pl.BlockSpec((ROWBLK, HIDDEN), blk),
            pl.BlockSpec((ROWBLK, HIDDEN), blk),
        ],
        out_shape=[jax.ShapeDtypeStruct((N_NODES, HIDDEN), jnp.float32)] * 3,
    )(x, wqT, wkT, wvT, wg1T, bg1, wg2, bg2)


def _pass1_body(qg_hbm, k_hbm, row_hbm, col_hbm, colh_hbm, z_hbm,
                scores_hbm, mpart_hbm,
                rowb, colb, qb, kb, sbh, idx0, idx1, idx2, idx3,
                mtmp, m_sh, sem1, sem2):
    idxbs = (idx0, idx1, idx2, idx3)
    cid = lax.axis_index("c")
    sid = lax.axis_index("s")
    wid = cid * NS + sid

    @pl.when(sid == 0)
    def _():
        pltpu.sync_copy(z_hbm, mtmp)
        pltpu.sync_copy(mtmp, m_sh)
    plsc.subcore_barrier()

    iota = lax.iota(jnp.int32, 16)
    ebase = wid * EPW

    def chunk_body(ci, carry):
        base = ebase + ci * CHUNK
        pltpu.sync_copy(row_hbm.at[pl.ds(base, CHUNK)], rowb)
        pltpu.sync_copy(col_hbm.at[pl.ds(base, CHUNK)], colb)
        for h in range(HEADS):
            pltpu.sync_copy(colh_hbm.at[pl.ds(h * N_EDGES + base, CHUNK)],
                            idxbs[h])
        d1 = pltpu.async_copy(qg_hbm.at[rowb], qb, sem1)
        d2 = pltpu.async_copy(k_hbm.at[colb], kb, sem2)
        d1.wait()
        d2.wait()

        def grp_body(g, c2):
            ev = g * 16 + iota
            for h in range(HEADS):
                acc = jnp.zeros((16,), jnp.float32)
                for dd in range(HEAD_DIM):
                    dv = jnp.full((16,), h * HEAD_DIM + dd, jnp.int32)
                    acc = acc + (plsc.load_gather(qb, [ev, dv])
                                 * plsc.load_gather(kb, [ev, dv]))
                sbh[pl.ds(h * CHUNK + g * 16, 16)] = acc
            return c2
        lax.fori_loop(0, GRP, grp_body, 0)

        pltpu.sync_copy(sbh, scores_hbm.at[pl.ds(base * HEADS, CHUNK * HEADS)])
        for h in range(HEADS):
            pltpu.sync_copy(sbh.at[pl.ds(h * CHUNK, CHUNK)],
                            m_sh.at[idxbs[h]],
                            add=True)
        return carry
    lax.fori_loop(0, NCHUNK, chunk_body, 0)

    plsc.subcore_barrier()

    @pl.when(sid == 0)
    def _():
        pltpu.sync_copy(m_sh, mtmp)
        pltpu.sync_copy(mtmp, mpart_hbm.at[pl.ds(cid * N_NODES * HEADS,
                                                 N_NODES * HEADS)])


def _pass1(qg, k, row, col, colh, zeros_flat):
    mesh = plsc.VectorSubcoreMesh(core_axis_name="c", subcore_axis_name="s")
    kern = functools.partial(
        pl.kernel,
        out_type=[jax.ShapeDtypeStruct((N_EDGES * HEADS,), jnp.float32),
                  jax.ShapeDtypeStruct((NC * N_NODES * HEADS,), jnp.float32)],
        mesh=mesh,
        scratch_types=[
            pltpu.VMEM((CHUNK,), jnp.int32),
            pltpu.VMEM((CHUNK,), jnp.int32),
            pltpu.VMEM((CHUNK, HIDDEN), jnp.float32),
            pltpu.VMEM((CHUNK, HIDDEN), jnp.float32),
            pltpu.VMEM((CHUNK * HEADS,), jnp.float32),
            pltpu.VMEM((CHUNK,), jnp.int32),
            pltpu.VMEM((CHUNK,), jnp.int32),
            pltpu.VMEM((CHUNK,), jnp.int32),
            pltpu.VMEM((CHUNK,), jnp.int32),
            pltpu.VMEM((N_NODES * HEADS,), jnp.float32),
            pltpu.VMEM_SHARED((N_NODES * HEADS,), jnp.float32),
            pltpu.SemaphoreType.DMA,
            pltpu.SemaphoreType.DMA,
        ],
        compiler_params=pltpu.CompilerParams(needs_layout_passes=False),
    )(_pass1_body)
    return kern(qg, k, row, col, colh, zeros_flat)


def _pass2_body(scores_hbm, col_hbm, colh_hbm, mpart_hbm, z_hbm,
                spart_hbm,
                colb, sbh, ebh, idx0, idx1, idx2, idx3, mv0, mv1, s_sh):
    idxbs = (idx0, idx1, idx2, idx3)
    cid = lax.axis_index("c")
    sid = lax.axis_index("s")
    wid = cid * NS + sid
    NH = N_NODES * HEADS

    @pl.when(sid == 0)
    def _():
        pltpu.sync_copy(z_hbm, mv0)
        pltpu.sync_copy(mv0, s_sh)
    pltpu.sync_copy(mpart_hbm.at[pl.ds(0, NH)], mv0)
    pltpu.sync_copy(mpart_hbm.at[pl.ds(NH, NH)], mv1)
    plsc.subcore_barrier()

    iota = lax.iota(jnp.int32, 16)
    ebase = wid * EPW

    def chunk_body(ci, carry):
        base = ebase + ci * CHUNK
        pltpu.sync_copy(col_hbm.at[pl.ds(base, CHUNK)], colb)
        for h in range(HEADS):
            pltpu.sync_copy(colh_hbm.at[pl.ds(h * N_EDGES + base, CHUNK)],
                            idxbs[h])
        pltpu.sync_copy(scores_hbm.at[pl.ds(base * HEADS, CHUNK * HEADS)], sbh)

        def grp_body(g, c2):
            colv4 = colb[pl.ds(g * 16, 16)] * HEADS
            for h in range(HEADS):
                sv = sbh[pl.ds(h * CHUNK + g * 16, 16)]
                m0 = plsc.load_gather(mv0, [colv4 + h])
                m1 = plsc.load_gather(mv1, [colv4 + h])
                e = jnp.exp(sv - m0 - m1)
                ebh[pl.ds(h * CHUNK + g * 16, 16)] = e
            return c2
        lax.fori_loop(0, GRP, grp_body, 0)

        for h in range(HEADS):
            pltpu.sync_copy(ebh.at[pl.ds(h * CHUNK, CHUNK)],
                            s_sh.at[idxbs[h]],
                            add=True)
        return carry
    lax.fori_loop(0, NCHUNK, chunk_body, 0)

    plsc.subcore_barrier()

    @pl.when(sid == 0)
    def _():
        pltpu.sync_copy(s_sh, mv0)
        pltpu.sync_copy(mv0, spart_hbm.at[pl.ds(cid * NH, NH)])


def _pass2(scores, col, colh, mpart, zeros_flat):
    mesh = plsc.VectorSubcoreMesh(core_axis_name="c", subcore_axis_name="s")
    kern = functools.partial(
        pl.kernel,
        out_type=jax.ShapeDtypeStruct((NC * N_NODES * HEADS,), jnp.float32),
        mesh=mesh,
        scratch_types=[
            pltpu.VMEM((CHUNK,), jnp.int32),
            pltpu.VMEM((CHUNK * HEADS,), jnp.float32),
            pltpu.VMEM((CHUNK * HEADS,), jnp.float32),
            pltpu.VMEM((CHUNK,), jnp.int32),
            pltpu.VMEM((CHUNK,), jnp.int32),
            pltpu.VMEM((CHUNK,), jnp.int32),
            pltpu.VMEM((CHUNK,), jnp.int32),
            pltpu.VMEM((N_NODES * HEADS,), jnp.float32),
            pltpu.VMEM((N_NODES * HEADS,), jnp.float32),
            pltpu.VMEM_SHARED((N_NODES * HEADS,), jnp.float32),
        ],
        compiler_params=pltpu.CompilerParams(needs_layout_passes=False),
    )(_pass2_body)
    return kern(scores, col, colh, mpart, zeros_flat)


def _tail_body(v_ref, s0_ref, s1_ref, x_ref, wo, bo, out_ref):
    s = s0_ref[...] + s1_ref[...]
    f = s / (s + 1e-8)
    r = lax.broadcasted_iota(jnp.int32, (HEADS, HIDDEN), 0)
    c = lax.broadcasted_iota(jnp.int32, (HEADS, HIDDEN), 1)
    bmat = (c // HEAD_DIM == r).astype(jnp.float32)
    fb = jnp.dot(f, bmat, preferred_element_type=jnp.float32)
    out_ref[...] = (jnp.dot(v_ref[...] * fb, wo[...],
                            preferred_element_type=jnp.float32)
                    + bo[...] + x_ref[...])


def _tail(v, s0, s1, x, woT, bo):
    nblk = N_NODES // ROWBLK
    full = lambda i: (0, 0)
    blk = lambda i: (i, 0)
    return pl.pallas_call(
        _tail_body,
        grid=(nblk,),
        in_specs=[
            pl.BlockSpec((ROWBLK, HIDDEN), blk),
            pl.BlockSpec((ROWBLK, HEADS), blk),
            pl.BlockSpec((ROWBLK, HEADS), blk),
            pl.BlockSpec((ROWBLK, HIDDEN), blk),
            pl.BlockSpec((HIDDEN, HIDDEN), full),
            pl.BlockSpec((1, HIDDEN), full),
        ],
        out_specs=pl.BlockSpec((ROWBLK, HIDDEN), blk),
        out_shape=jax.ShapeDtypeStruct((N_NODES, HIDDEN), jnp.float32),
    )(v, s0, s1, x, woT, bo)


def kernel(x, edge_index, W_q, W_k, W_v, W_o, b_o, W_g1, b_g1, W_g2, b_g2):
    qg, k, v = _prep(x, W_q.T, W_k.T, W_v.T, W_g1.T,
                     b_g1.reshape(1, -1), W_g2, b_g2.reshape(1, 1))
    row = edge_index[0]
    col = edge_index[1]
    colh = (col[None, :] * HEADS
            + jnp.arange(HEADS, dtype=jnp.int32)[:, None]).reshape(-1)
    zeros_flat = jnp.zeros((N_NODES * HEADS,), jnp.float32)
    scores, mpart = _pass1(qg, k, row, col, colh, zeros_flat)
    spart = _pass2(scores, col, colh, mpart, zeros_flat)
    spart = spart.reshape(NC, N_NODES, HEADS)
    return _tail(v, spart[0], spart[1], x, W_o.T, b_o.reshape(1, -1))
